# baseline (speedup 1.0000x reference)
import functools

import jax
import jax.numpy as jnp
from jax import lax
from jax.experimental import pallas as pl
from jax.experimental.pallas import tpu as pltpu
from jax.experimental.pallas import tpu_sc as plsc

N = 10000
E = 320000
D = 128
H = 32
G = 64
C = 2

W = 48
NCORES = 2
NSUB = 16
NTILES = NCORES * NSUB
KC = 128
GS = 3
NCHUNKS_TOT = E // KC
NCHUNK = NCHUNKS_TOT // NTILES
NEXTRA = NCHUNKS_TOT - NCHUNK * NTILES
NG = NCHUNK // GS
NG2 = NG // 2
RPT = 640
NPAD = NSUB * RPT
GROWS = GS * KC

_HIGH = lax.Precision.HIGHEST


NB1 = 10
RB1 = N // NB1


def _proj1_body(x_ref, w_ref, ya_ref):
    yl = lax.dot_general(x_ref[...], w_ref[...], (((1,), (0,)), ((), ())),
                         preferred_element_type=jnp.float32, precision=_HIGH)
    ya_ref[...] = jnp.concatenate(
        [yl, jnp.ones((RB1, 1), jnp.float32),
         jnp.zeros((RB1, W - H - 1), jnp.float32)], axis=1)


def _proj2_body(x_ref, w_ref, yrt_ref):
    yrt_ref[...] = lax.dot_general(w_ref[...], x_ref[...],
                                   (((0,), (1,)), ((), ())),
                                   preferred_element_type=jnp.float32,
                                   precision=_HIGH)


def _project1(x, w_l):
    return pl.pallas_call(
        _proj1_body,
        grid=(NB1,),
        in_specs=[pl.BlockSpec((RB1, D), lambda i: (i, 0)),
                  pl.BlockSpec((D, H), lambda i: (0, 0))],
        out_specs=[pl.BlockSpec((RB1, W), lambda i: (i, 0))],
        out_shape=[jax.ShapeDtypeStruct((N, W), jnp.float32)],
    )(x, w_l)


def _project2(x, w_r):
    return pl.pallas_call(
        _proj2_body,
        out_shape=[jax.ShapeDtypeStruct((H, N), jnp.float32)],
    )(x, w_r)


def _sc_agg_body(ya_hbm, ei_hbm, acc_hbm,
                 src_v, dst_v, rows_a, rows_b, acc,
                 sem_ga, sem_gb, sem_za, sem_zb):
    cid = lax.axis_index("c")
    sid = lax.axis_index("s")
    wid = cid * NSUB + sid
    rbase = sid * RPT
    cbase = wid * NCHUNK

    def _gathers(g, buf, sem):
        return [pltpu.make_async_copy(
            ya_hbm.at[src_v.at[g * GS + b]],
            buf.at[pl.ds(b * KC, KC)], sem) for b in range(GS)]

    def _scatters(g, buf, sem):
        return [pltpu.make_async_copy(
            buf.at[pl.ds(b * KC, KC)],
            acc.at[dst_v.at[g * GS + b]], sem) for b in range(GS)]

    zero16 = jnp.zeros((16,), jnp.float32)

    def _zero_row(i, carry):
        for c in range(W // 16):
            rows_a[i, pl.ds(c * 16, 16)] = zero16
        return carry

    lax.fori_loop(0, GROWS, _zero_row, 0)
    pltpu.sync_copy(rows_a, acc.at[pl.ds(rbase, GROWS)])
    pltpu.sync_copy(rows_a.at[pl.ds(0, RPT - GROWS)],
                    acc.at[pl.ds(rbase + GROWS, RPT - GROWS)])

    pltpu.sync_copy(ei_hbm.at[0, pl.ds(cbase, NCHUNK)],
                    src_v.at[pl.ds(0, NCHUNK)])
    pltpu.sync_copy(ei_hbm.at[1, pl.ds(cbase, NCHUNK)],
                    dst_v.at[pl.ds(0, NCHUNK)])

    @pl.when(wid < NEXTRA)
    def _():
        pltpu.sync_copy(ei_hbm.at[0, pl.ds(NTILES * NCHUNK + wid, 1)],
                        src_v.at[pl.ds(NCHUNK, 1)])
        pltpu.sync_copy(ei_hbm.at[1, pl.ds(NTILES * NCHUNK + wid, 1)],
                        dst_v.at[pl.ds(NCHUNK, 1)])
    plsc.subcore_barrier()

    for d in _gathers(0, rows_a, sem_ga):
        d.start()

    def _pair(i, carry):
        ga = 2 * i
        gb = 2 * i + 1

        @pl.when(i > 0)
        def _():
            for d in _scatters(gb - 2, rows_b, sem_zb):
                d.wait()

        for d in _gathers(gb, rows_b, sem_gb):
            d.start()
        for d in _gathers(ga, rows_a, sem_ga):
            d.wait()
        for d in _scatters(ga, rows_a, sem_za):
            d.start(add=True)
        for d in _scatters(ga, rows_a, sem_za):
            d.wait()

        @pl.when(i < NG2 - 1)
        def _():
            for d in _gathers(ga + 2, rows_a, sem_ga):
                d.start()

        for d in _gathers(gb, rows_b, sem_gb):
            d.wait()
        for d in _scatters(gb, rows_b, sem_zb):
            d.start(add=True)
        return carry

    lax.fori_loop(0, NG2, _pair, 0)
    for d in _scatters(NG - 1, rows_b, sem_zb):
        d.wait()

    @pl.when(wid < NEXTRA)
    def _():
        pltpu.async_copy(ya_hbm.at[src_v.at[NCHUNK]],
                         rows_a.at[pl.ds(0, KC)], sem_ga).wait()
        d = pltpu.make_async_copy(rows_a.at[pl.ds(0, KC)],
                                  acc.at[dst_v.at[NCHUNK]], sem_za)
        d.start(add=True)
        d.wait()

    plsc.subcore_barrier()

    pltpu.sync_copy(acc.at[pl.ds(rbase, GROWS)], rows_a)
    pltpu.sync_copy(rows_a, acc_hbm.at[cid, pl.ds(rbase, GROWS)])
    pltpu.sync_copy(acc.at[pl.ds(rbase + GROWS, RPT - GROWS)],
                    rows_b.at[pl.ds(0, RPT - GROWS)])
    pltpu.sync_copy(rows_b.at[pl.ds(0, RPT - GROWS)],
                    acc_hbm.at[cid, pl.ds(rbase + GROWS, RPT - GROWS)])


@functools.cache
def _sc_agg():
    mesh = plsc.VectorSubcoreMesh(core_axis_name="c", subcore_axis_name="s")
    return pl.kernel(
        _sc_agg_body,
        mesh=mesh,
        compiler_params=pltpu.CompilerParams(use_tc_tiling_on_sc=False),
        out_type=[jax.ShapeDtypeStruct((NCORES, NPAD, W), jnp.float32)],
        scratch_types=[
            pltpu.VMEM((NCHUNK + 1, KC), jnp.int32),
            pltpu.VMEM((NCHUNK + 1, KC), jnp.int32),
            pltpu.VMEM((GROWS, W), jnp.float32),
            pltpu.VMEM((GROWS, W), jnp.float32),
            pltpu.VMEM_SHARED((NPAD, W), jnp.float32),
            pltpu.SemaphoreType.DMA,
            pltpu.SemaphoreType.DMA,
            pltpu.SemaphoreType.DMA,
            pltpu.SemaphoreType.DMA,
        ],
    )


def _head_body(apt_ref, yrt_ref, batch_ref, bl_ref, ga_ref, be_ref,
               w1_ref, b1_ref, w2_ref, b2_ref, wa_ref, ba_ref,
               out_ref, an_ref):
    at = jnp.transpose(apt_ref[0] + apt_ref[1], (1, 0))
    z = at[:H, :N]
    cnt = at[H:H + 1, :N]
    h = z / jnp.maximum(cnt, 1.0) + yrt_ref[...] + bl_ref[...]
    mean = jnp.mean(h, axis=1, keepdims=True)
    var = jnp.mean((h - mean) ** 2, axis=1, keepdims=True)
    hn = (h - mean) * lax.rsqrt(var + 1e-5) * ga_ref[...] + be_ref[...]
    hr = jnp.maximum(hn, 0.0)

    gid = lax.broadcasted_iota(jnp.int32, (N, G), 1)
    p = (batch_ref[...] == gid).astype(jnp.float32)
    psum = lax.dot_general(hr, p, (((1,), (0,)), ((), ())),
                           preferred_element_type=jnp.float32)
    gcnt = lax.dot_general(jnp.ones((1, N), jnp.float32), p,
                           (((1,), (0,)), ((), ())),
                           preferred_element_type=jnp.float32)
    pooled = psum / jnp.maximum(gcnt, 1.0)

    hid = jnp.maximum(
        lax.dot_general(w1_ref[...], pooled, (((0,), (0,)), ((), ())),
                        preferred_element_type=jnp.float32, precision=_HIGH)
        + b1_ref[...], 0.0)
    out_ref[...] = lax.dot_general(hid, w2_ref[...], (((0,), (0,)), ((), ())),
                                   preferred_element_type=jnp.float32,
                                   precision=_HIGH) + b2_ref[...]
    sa = lax.dot_general(pooled, wa_ref[...], (((0,), (0,)), ((), ())),
                         preferred_element_type=jnp.float32,
                         precision=_HIGH) + ba_ref[...]
    an_ref[...] = 1.0 / (1.0 + jnp.exp(-sa))


def _heads(apt, yrt, batch2, b_l, gamma, beta, W1, b1, W2, b2, Wa, ba):
    return pl.pallas_call(
        _head_body,
        out_shape=[jax.ShapeDtypeStruct((G, C), jnp.float32),
                   jax.ShapeDtypeStruct((G, 1), jnp.float32)],
    )(apt, yrt, batch2, b_l, gamma, beta, W1, b1, W2, b2, Wa, ba)


def kernel(x, edge_index, batch, W_l, b_l, W_r, gamma, beta,
           W1, b1, W2, b2, Wa, ba):
    (ya,) = _project1(x, W_l)
    (yrt,) = _project2(x, W_r)

    ei = edge_index.reshape(2, NCHUNKS_TOT, KC)

    (ap,) = _sc_agg()(ya, ei)

    out, an = _heads(
        ap, yrt, batch.reshape(N, 1),
        b_l.reshape(H, 1), gamma.reshape(H, 1), beta.reshape(H, 1),
        W1, b1.reshape(16, 1), W2, b2.reshape(1, C), Wa, ba.reshape(1, 1))
    return (out, an)


# speedup vs baseline: 1.0403x; 1.0403x over previous
import functools

import jax
import jax.numpy as jnp
from jax import lax
from jax.experimental import pallas as pl
from jax.experimental.pallas import tpu as pltpu
from jax.experimental.pallas import tpu_sc as plsc

N = 10000
E = 320000
D = 128
H = 32
G = 64
C = 2

W = 48
NCORES = 2
NSUB = 16
NTILES = NCORES * NSUB
KC = 128
GS = 3
NCHUNKS_TOT = E // KC
NCHUNK = NCHUNKS_TOT // NTILES
NEXTRA = NCHUNKS_TOT - NCHUNK * NTILES
NG = NCHUNK // GS
NG2 = NG // 2
RPT = 640
NPAD = NSUB * RPT
GROWS = GS * KC

_HIGH = lax.Precision.HIGHEST


def _proj1_body(x_ref, w_ref, ya_ref):
    yl = lax.dot_general(x_ref[...], w_ref[...], (((1,), (0,)), ((), ())),
                         preferred_element_type=jnp.float32, precision=_HIGH)
    ya_ref[...] = jnp.concatenate(
        [yl, jnp.ones((N, 1), jnp.float32),
         jnp.zeros((N, W - H - 1), jnp.float32)], axis=1)


def _proj2_body(x_ref, w_ref, yrt_ref):
    yrt_ref[...] = lax.dot_general(w_ref[...], x_ref[...],
                                   (((0,), (1,)), ((), ())),
                                   preferred_element_type=jnp.float32,
                                   precision=_HIGH)


def _project1(x, w_l):
    return pl.pallas_call(
        _proj1_body,
        out_shape=[jax.ShapeDtypeStruct((N, W), jnp.float32)],
    )(x, w_l)


def _project2(x, w_r):
    return pl.pallas_call(
        _proj2_body,
        out_shape=[jax.ShapeDtypeStruct((H, N), jnp.float32)],
    )(x, w_r)


def _sc_agg_body(ya_hbm, ei_hbm, acc_hbm,
                 src_v, dst_v, rows_a, rows_b, acc,
                 sem_ga, sem_gb, sem_za, sem_zb):
    cid = lax.axis_index("c")
    sid = lax.axis_index("s")
    wid = cid * NSUB + sid
    rbase = sid * RPT
    cbase = wid * NCHUNK

    def _gathers(g, buf, sem):
        return [pltpu.make_async_copy(
            ya_hbm.at[src_v.at[g * GS + b]],
            buf.at[pl.ds(b * KC, KC)], sem) for b in range(GS)]

    def _scatters(g, buf, sem):
        return [pltpu.make_async_copy(
            buf.at[pl.ds(b * KC, KC)],
            acc.at[dst_v.at[g * GS + b]], sem) for b in range(GS)]

    zero16 = jnp.zeros((16,), jnp.float32)

    def _zero_row(i, carry):
        for c in range(W // 16):
            rows_a[i, pl.ds(c * 16, 16)] = zero16
        return carry

    lax.fori_loop(0, GROWS, _zero_row, 0)
    pltpu.sync_copy(rows_a, acc.at[pl.ds(rbase, GROWS)])
    pltpu.sync_copy(rows_a.at[pl.ds(0, RPT - GROWS)],
                    acc.at[pl.ds(rbase + GROWS, RPT - GROWS)])

    pltpu.sync_copy(ei_hbm.at[0, pl.ds(cbase, NCHUNK)],
                    src_v.at[pl.ds(0, NCHUNK)])
    pltpu.sync_copy(ei_hbm.at[1, pl.ds(cbase, NCHUNK)],
                    dst_v.at[pl.ds(0, NCHUNK)])

    @pl.when(wid < NEXTRA)
    def _():
        pltpu.sync_copy(ei_hbm.at[0, pl.ds(NTILES * NCHUNK + wid, 1)],
                        src_v.at[pl.ds(NCHUNK, 1)])
        pltpu.sync_copy(ei_hbm.at[1, pl.ds(NTILES * NCHUNK + wid, 1)],
                        dst_v.at[pl.ds(NCHUNK, 1)])
    plsc.subcore_barrier()

    for d in _gathers(0, rows_a, sem_ga):
        d.start()

    def _pair(i, carry):
        ga = 2 * i
        gb = 2 * i + 1

        @pl.when(i > 0)
        def _():
            for d in _scatters(gb - 2, rows_b, sem_zb):
                d.wait()

        for d in _gathers(gb, rows_b, sem_gb):
            d.start()
        for d in _gathers(ga, rows_a, sem_ga):
            d.wait()
        for d in _scatters(ga, rows_a, sem_za):
            d.start(add=True)
        for d in _scatters(ga, rows_a, sem_za):
            d.wait()

        @pl.when(i < NG2 - 1)
        def _():
            for d in _gathers(ga + 2, rows_a, sem_ga):
                d.start()

        for d in _gathers(gb, rows_b, sem_gb):
            d.wait()
        for d in _scatters(gb, rows_b, sem_zb):
            d.start(add=True)
        return carry

    lax.fori_loop(0, NG2, _pair, 0)
    for d in _scatters(NG - 1, rows_b, sem_zb):
        d.wait()

    @pl.when(wid < NEXTRA)
    def _():
        pltpu.async_copy(ya_hbm.at[src_v.at[NCHUNK]],
                         rows_a.at[pl.ds(0, KC)], sem_ga).wait()
        d = pltpu.make_async_copy(rows_a.at[pl.ds(0, KC)],
                                  acc.at[dst_v.at[NCHUNK]], sem_za)
        d.start(add=True)
        d.wait()

    plsc.subcore_barrier()

    pltpu.sync_copy(acc.at[pl.ds(rbase, GROWS)], rows_a)
    pltpu.sync_copy(rows_a, acc_hbm.at[cid, pl.ds(rbase, GROWS)])
    pltpu.sync_copy(acc.at[pl.ds(rbase + GROWS, RPT - GROWS)],
                    rows_b.at[pl.ds(0, RPT - GROWS)])
    pltpu.sync_copy(rows_b.at[pl.ds(0, RPT - GROWS)],
                    acc_hbm.at[cid, pl.ds(rbase + GROWS, RPT - GROWS)])


@functools.cache
def _sc_agg():
    mesh = plsc.VectorSubcoreMesh(core_axis_name="c", subcore_axis_name="s")
    return pl.kernel(
        _sc_agg_body,
        mesh=mesh,
        compiler_params=pltpu.CompilerParams(use_tc_tiling_on_sc=False),
        out_type=[jax.ShapeDtypeStruct((NCORES, NPAD, W), jnp.float32)],
        scratch_types=[
            pltpu.VMEM((NCHUNK + 1, KC), jnp.int32),
            pltpu.VMEM((NCHUNK + 1, KC), jnp.int32),
            pltpu.VMEM((GROWS, W), jnp.float32),
            pltpu.VMEM((GROWS, W), jnp.float32),
            pltpu.VMEM_SHARED((NPAD, W), jnp.float32),
            pltpu.SemaphoreType.DMA,
            pltpu.SemaphoreType.DMA,
            pltpu.SemaphoreType.DMA,
            pltpu.SemaphoreType.DMA,
        ],
    )


def _head_body(apt_ref, yrt_ref, batch_ref, bl_ref, ga_ref, be_ref,
               w1_ref, b1_ref, w2_ref, b2_ref, wa_ref, ba_ref,
               out_ref, an_ref):
    at = jnp.transpose(apt_ref[0] + apt_ref[1], (1, 0))
    z = at[:H, :N]
    cnt = at[H:H + 1, :N]
    h = z / jnp.maximum(cnt, 1.0) + yrt_ref[...] + bl_ref[...]
    mean = jnp.mean(h, axis=1, keepdims=True)
    var = jnp.mean((h - mean) ** 2, axis=1, keepdims=True)
    hn = (h - mean) * lax.rsqrt(var + 1e-5) * ga_ref[...] + be_ref[...]
    hr = jnp.maximum(hn, 0.0)

    gid = lax.broadcasted_iota(jnp.int32, (N, G), 1)
    p = (batch_ref[...] == gid).astype(jnp.float32)
    psum = lax.dot_general(hr, p, (((1,), (0,)), ((), ())),
                           preferred_element_type=jnp.float32)
    gcnt = lax.dot_general(jnp.ones((1, N), jnp.float32), p,
                           (((1,), (0,)), ((), ())),
                           preferred_element_type=jnp.float32)
    pooled = psum / jnp.maximum(gcnt, 1.0)

    hid = jnp.maximum(
        lax.dot_general(w1_ref[...], pooled, (((0,), (0,)), ((), ())),
                        preferred_element_type=jnp.float32, precision=_HIGH)
        + b1_ref[...], 0.0)
    out_ref[...] = lax.dot_general(hid, w2_ref[...], (((0,), (0,)), ((), ())),
                                   preferred_element_type=jnp.float32,
                                   precision=_HIGH) + b2_ref[...]
    sa = lax.dot_general(pooled, wa_ref[...], (((0,), (0,)), ((), ())),
                         preferred_element_type=jnp.float32,
                         precision=_HIGH) + ba_ref[...]
    an_ref[...] = 1.0 / (1.0 + jnp.exp(-sa))


def _heads(apt, yrt, batch2, b_l, gamma, beta, W1, b1, W2, b2, Wa, ba):
    return pl.pallas_call(
        _head_body,
        out_shape=[jax.ShapeDtypeStruct((G, C), jnp.float32),
                   jax.ShapeDtypeStruct((G, 1), jnp.float32)],
    )(apt, yrt, batch2, b_l, gamma, beta, W1, b1, W2, b2, Wa, ba)


def kernel(x, edge_index, batch, W_l, b_l, W_r, gamma, beta,
           W1, b1, W2, b2, Wa, ba):
    (ya,) = _project1(x, W_l)
    (yrt,) = _project2(x, W_r)

    ei = edge_index.reshape(2, NCHUNKS_TOT, KC)

    (ap,) = _sc_agg()(ya, ei)

    out, an = _heads(
        ap, yrt, batch.reshape(N, 1),
        b_l.reshape(H, 1), gamma.reshape(H, 1), beta.reshape(H, 1),
        W1, b1.reshape(16, 1), W2, b2.reshape(1, C), Wa, ba.reshape(1, 1))
    return (out, an)
